# baseline (speedup 1.0000x reference)
import functools

import jax
import jax.numpy as jnp
from jax import lax
from jax.experimental import pallas as pl
from jax.experimental.pallas import tpu as pltpu
from jax.experimental.pallas import tpu_sc as plsc

NC = 2
NS = 16
NW = NC * NS
K = 1024


def _round_up(x, m):
    return (x + m - 1) // m * m


@functools.partial(jax.jit, static_argnums=(4, 5))
def _gather_sc(p0_flat, p1_flat, idx0p, idx1p, pw0, n1p):
    B0 = p0_flat.shape[0]
    B1 = p1_flat.shape[0]
    mesh = plsc.VectorSubcoreMesh(core_axis_name="c", subcore_axis_name="s")

    @functools.partial(
        pl.kernel,
        mesh=mesh,
        out_type=[
            jax.ShapeDtypeStruct((B0, NW * pw0), jnp.float32),
            jax.ShapeDtypeStruct((B1, n1p), jnp.float32),
        ],
        scratch_types=[
            pltpu.VMEM((pw0,), jnp.int32),
            pltpu.VMEM((K,), jnp.float32),
            pltpu.VMEM((n1p,), jnp.int32),
            pltpu.VMEM((n1p,), jnp.float32),
            pltpu.SemaphoreType.DMA,
        ],
        compiler_params=pltpu.CompilerParams(use_tc_tiling_on_sc=False),
    )
    def body(p0, p1, i0, i1, out0, out1, idx_v, vals_v, idx1_v, vals1_v, sem):
        wid = lax.axis_index("c") * NS + lax.axis_index("s")
        base = wid * pw0
        pltpu.sync_copy(i0.at[pl.ds(base, pw0)], idx_v)

        for b in range(B0):
            def chunk_body(c, _):
                pltpu.async_copy(
                    p0.at[b].at[idx_v.at[pl.ds(c * K, K)]], vals_v, sem
                ).wait()
                pltpu.sync_copy(vals_v, out0.at[b].at[pl.ds(base + c * K, K)])
                return 0

            lax.fori_loop(0, pw0 // K, chunk_body, 0)

        @pl.when(wid < B1)
        def _small():
            pltpu.sync_copy(i1.at[pl.ds(0, n1p)], idx1_v)
            pltpu.async_copy(p1.at[wid].at[idx1_v], vals1_v, sem).wait()
            pltpu.sync_copy(vals1_v, out1.at[wid])

    return body(p0_flat, p1_flat, idx0p, idx1p)


def kernel(params_0, params_1, idx_0, idx_1):
    B0 = params_0.shape[0]
    B1 = params_1.shape[0]
    n0 = idx_0.shape[0]
    n1 = idx_1.shape[0]
    pw0 = _round_up(_round_up(n0, NW) // NW, K)
    n1p = _round_up(n1, 128)

    p0_flat = params_0.reshape(B0, -1)
    idx0p = jnp.zeros((NW * pw0,), jnp.int32).at[:n0].set(idx_0.astype(jnp.int32))
    idx1p = jnp.zeros((n1p,), jnp.int32).at[:n1].set(idx_1.astype(jnp.int32))

    out0p, out1p = _gather_sc(p0_flat, params_1, idx0p, idx1p, pw0, n1p)
    return (out0p[:, :n0], out1p[:, :n1])


# speedup vs baseline: 1.9806x; 1.9806x over previous
import functools

import jax
import jax.numpy as jnp
from jax import lax
from jax.experimental import pallas as pl
from jax.experimental.pallas import tpu as pltpu
from jax.experimental.pallas import tpu_sc as plsc

NC = 2
NS = 16
NW = NC * NS


def _round_up(x, m):
    return (x + m - 1) // m * m


@functools.partial(jax.jit, static_argnums=(4, 5))
def _gather_sc(p0_flat, p1_flat, idx0p, idx1p, pw0, n1p):
    B0 = p0_flat.shape[0]
    B1 = p1_flat.shape[0]
    mesh = plsc.VectorSubcoreMesh(core_axis_name="c", subcore_axis_name="s")

    @functools.partial(
        pl.kernel,
        mesh=mesh,
        out_type=[
            jax.ShapeDtypeStruct((B0, NW * pw0), jnp.float32),
            jax.ShapeDtypeStruct((B1, n1p), jnp.float32),
        ],
        scratch_types=[
            pltpu.VMEM((pw0,), jnp.int32),
            pltpu.VMEM((pw0,), jnp.float32),
            pltpu.VMEM((pw0,), jnp.float32),
            pltpu.VMEM((n1p,), jnp.int32),
            pltpu.VMEM((n1p,), jnp.float32),
            pltpu.SemaphoreType.DMA,
            pltpu.SemaphoreType.DMA,
        ],
        compiler_params=pltpu.CompilerParams(use_tc_tiling_on_sc=False),
    )
    def body(p0, p1, i0, i1, out0, out1, idx_v, val0_v, val1_v, idx1_v,
             vals1_v, gsem, ssem):
        wid = lax.axis_index("c") * NS + lax.axis_index("s")
        base = wid * pw0
        pltpu.sync_copy(i0.at[pl.ds(base, pw0)], idx_v)

        bufs = (val0_v, val1_v)
        gathers = [None] * B0
        stores = [None] * B0
        gathers[0] = pltpu.async_copy(p0.at[0].at[idx_v], bufs[0], gsem)
        for b in range(B0):
            gathers[b].wait()
            if b + 1 < B0:
                if b >= 1:
                    stores[b - 1].wait()
                gathers[b + 1] = pltpu.async_copy(
                    p0.at[b + 1].at[idx_v], bufs[(b + 1) % 2], gsem)
            stores[b] = pltpu.async_copy(
                bufs[b % 2], out0.at[b].at[pl.ds(base, pw0)], ssem)
        stores[B0 - 2].wait()
        stores[B0 - 1].wait()

        @pl.when(wid < B1)
        def _small():
            pltpu.sync_copy(i1.at[pl.ds(0, n1p)], idx1_v)
            pltpu.async_copy(p1.at[wid].at[idx1_v], vals1_v, gsem).wait()
            pltpu.sync_copy(vals1_v, out1.at[wid])

    return body(p0_flat, p1_flat, idx0p, idx1p)


def kernel(params_0, params_1, idx_0, idx_1):
    B0 = params_0.shape[0]
    B1 = params_1.shape[0]
    n0 = idx_0.shape[0]
    n1 = idx_1.shape[0]
    n_elem0 = params_0.size // B0
    pw0 = _round_up(_round_up(n0, NW) // NW, 128)
    n1p = _round_up(n1, 128)
    pad0 = NW * pw0 - n0

    p0_flat = params_0.reshape(B0, -1)
    fill0 = (jnp.arange(pad0, dtype=jnp.int32) * 16) % n_elem0
    idx0p = jnp.concatenate([idx_0.astype(jnp.int32), fill0])
    idx1p = jnp.zeros((n1p,), jnp.int32).at[:n1].set(idx_1.astype(jnp.int32))

    out0p, out1p = _gather_sc(p0_flat, params_1, idx0p, idx1p, pw0, n1p)
    return (out0p[:, :n0], out1p[:, :n1])


# speedup vs baseline: 2.0112x; 1.0155x over previous
import functools

import jax
import jax.numpy as jnp
from jax import lax
from jax.experimental import pallas as pl
from jax.experimental.pallas import tpu as pltpu
from jax.experimental.pallas import tpu_sc as plsc

NC = 2
NS = 16
NW = NC * NS


def _round_up(x, m):
    return (x + m - 1) // m * m


@functools.partial(jax.jit, static_argnums=(4, 5))
def _gather_sc(p0_flat, p1_flat, idx0p, idx1p, pw0, n1p):
    B0 = p0_flat.shape[0]
    B1 = p1_flat.shape[0]
    mesh = plsc.VectorSubcoreMesh(core_axis_name="c", subcore_axis_name="s")

    @functools.partial(
        pl.kernel,
        mesh=mesh,
        out_type=[
            jax.ShapeDtypeStruct((B0, NW * pw0), jnp.float32),
            jax.ShapeDtypeStruct((B1, n1p), jnp.float32),
        ],
        scratch_types=[
            pltpu.VMEM((pw0,), jnp.int32),
        ] + [pltpu.VMEM((pw0,), jnp.float32) for _ in range(B0)] + [
            pltpu.VMEM((n1p,), jnp.int32),
            pltpu.VMEM((n1p,), jnp.float32),
            pltpu.SemaphoreType.DMA,
            pltpu.SemaphoreType.DMA,
        ],
        compiler_params=pltpu.CompilerParams(use_tc_tiling_on_sc=False),
    )
    def body(p0, p1, i0, i1, out0, out1, idx_v, *rest):
        bufs = rest[:B0]
        idx1_v, vals1_v, gsem, ssem = rest[B0:]
        wid = lax.axis_index("c") * NS + lax.axis_index("s")
        base = wid * pw0
        pltpu.sync_copy(i0.at[pl.ds(base, pw0)], idx_v)

        gathers = [
            pltpu.async_copy(p0.at[b].at[idx_v], bufs[b], gsem)
            for b in range(B0)
        ]
        stores = []
        for b in range(B0):
            gathers[b].wait()
            stores.append(pltpu.async_copy(
                bufs[b], out0.at[b].at[pl.ds(base, pw0)], ssem))
        for st in stores:
            st.wait()

        @pl.when(wid < B1)
        def _small():
            pltpu.sync_copy(i1.at[pl.ds(0, n1p)], idx1_v)
            pltpu.async_copy(p1.at[wid].at[idx1_v], vals1_v, gsem).wait()
            pltpu.sync_copy(vals1_v, out1.at[wid])

    return body(p0_flat, p1_flat, idx0p, idx1p)


def kernel(params_0, params_1, idx_0, idx_1):
    B0 = params_0.shape[0]
    B1 = params_1.shape[0]
    n0 = idx_0.shape[0]
    n1 = idx_1.shape[0]
    n_elem0 = params_0.size // B0
    pw0 = _round_up(_round_up(n0, NW) // NW, 128)
    n1p = _round_up(n1, 128)
    pad0 = NW * pw0 - n0

    p0_flat = params_0.reshape(B0, -1)
    fill0 = (jnp.arange(pad0, dtype=jnp.int32) * 16) % n_elem0
    idx0p = jnp.concatenate([idx_0.astype(jnp.int32), fill0])
    idx1p = jnp.zeros((n1p,), jnp.int32).at[:n1].set(idx_1.astype(jnp.int32))

    out0p, out1p = _gather_sc(p0_flat, params_1, idx0p, idx1p, pw0, n1p)
    return (out0p[:, :n0], out1p[:, :n1])
